# baseline (speedup 1.0000x reference)
import functools

import jax
import jax.numpy as jnp
from jax import lax
from jax.experimental import pallas as pl
from jax.experimental.pallas import tpu as pltpu
from jax.experimental.pallas import tpu_sc as plsc

NC = 2
NS = 16
NW = NC * NS
L = 16
CHUNK = 128


@functools.lru_cache(maxsize=None)
def _make_sc_gather(B, D, n_head, n_tail):
    assert B % (NW * L) == 0
    b_per_w = B // NW
    n_chunks = b_per_w // CHUNK
    assert n_chunks * CHUNK == b_per_w
    pad = 2 * NW

    mesh = plsc.VectorSubcoreMesh(core_axis_name="c", subcore_axis_name="s")

    scratch = [pltpu.VMEM((b_per_w,), jnp.int32)]
    scratch += [pltpu.VMEM((CHUNK,), jnp.int32) for _ in range(4 * n_chunks)]
    scratch += [pltpu.VMEM((CHUNK, D), jnp.float32) for _ in range(2 * n_chunks)]
    scratch += [pltpu.SemaphoreType.DMA, pltpu.SemaphoreType.DMA,
                pltpu.SemaphoreType.DMA]

    @functools.partial(
        pl.kernel,
        mesh=mesh,
        out_type=jax.ShapeDtypeStruct((B + pad, D), jnp.float32),
        scratch_types=scratch,
        compiler_params=pltpu.CompilerParams(use_tc_tiling_on_sc=False),
    )
    def k(head_hbm, tail_hbm, ids_hbm, out_hbm, *s):
        ids_v = s[0]
        o = 1
        idx_a = s[o:o + n_chunks]; o += n_chunks
        idx_b = s[o:o + n_chunks]; o += n_chunks
        pos_a = s[o:o + n_chunks]; o += n_chunks
        pos_b = s[o:o + n_chunks]; o += n_chunks
        buf_a = s[o:o + n_chunks]; o += n_chunks
        buf_b = s[o:o + n_chunks]; o += n_chunks
        sem_a, sem_b, sem_s = s[o:o + 3]

        wid = lax.axis_index("s") * NC + lax.axis_index("c")
        base = wid * b_per_w
        pltpu.sync_copy(ids_hbm.at[pl.ds(base, b_per_w)], ids_v)

        trash_a = B + 2 * wid
        trash_b = trash_a + 1
        iota = lax.iota(jnp.int32, L)
        for i in range(b_per_w // L):
            j, off = divmod(i * L, CHUNK)
            v = ids_v[pl.ds(i * L, L)]
            m = v >= n_head
            idx_a[j][pl.ds(off, L)] = jnp.minimum(v, n_head - 1)
            idx_b[j][pl.ds(off, L)] = jnp.maximum(v - n_head, 0)
            rows = base + i * L + iota
            pos_a[j][pl.ds(off, L)] = jnp.where(m, trash_a, rows)
            pos_b[j][pl.ds(off, L)] = jnp.where(m, rows, trash_b)

        gathers = []
        for j in range(n_chunks):
            gathers.append(pltpu.async_copy(head_hbm.at[idx_a[j]], buf_a[j], sem_a))
            gathers.append(pltpu.async_copy(tail_hbm.at[idx_b[j]], buf_b[j], sem_b))
        for g in gathers:
            g.wait()
        scatters = []
        for j in range(n_chunks):
            scatters.append(pltpu.async_copy(buf_a[j], out_hbm.at[pos_a[j]], sem_s))
            scatters.append(pltpu.async_copy(buf_b[j], out_hbm.at[pos_b[j]], sem_s))
        for t in scatters:
            t.wait()

    return k, pad


def kernel(weight_head, trainable_buffer, input_ids):
    n_head, D = weight_head.shape
    n_tail = trainable_buffer.shape[0]
    B = input_ids.shape[0]
    k, pad = _make_sc_gather(B, D, n_head, n_tail)
    out = k(weight_head, trainable_buffer, input_ids.astype(jnp.int32))
    return out[:B]


# speedup vs baseline: 1.0235x; 1.0235x over previous
import functools

import jax
import jax.numpy as jnp
from jax import lax
from jax.experimental import pallas as pl
from jax.experimental.pallas import tpu as pltpu
from jax.experimental.pallas import tpu_sc as plsc

NC = 2
NS = 16
NW = NC * NS
L = 16


@functools.lru_cache(maxsize=None)
def _make_sc_gather(B, D, n_head, n_tail):
    assert B % (NW * L) == 0
    b_per_w = B // NW
    b_per_c = NS * b_per_w
    pad = 2 * NS

    mesh = plsc.VectorSubcoreMesh(core_axis_name="c", subcore_axis_name="s")

    scratch = [
        pltpu.VMEM((b_per_w,), jnp.int32),
        pltpu.VMEM((b_per_w,), jnp.int32),
        pltpu.VMEM((b_per_w,), jnp.int32),
        pltpu.VMEM((b_per_w,), jnp.int32),
        pltpu.VMEM((b_per_w,), jnp.int32),
        pltpu.VMEM((b_per_w, D), jnp.float32),
        pltpu.VMEM((b_per_w, D), jnp.float32),
        pltpu.VMEM_SHARED((b_per_c + pad, D), jnp.float32),
        pltpu.SemaphoreType.DMA,
        pltpu.SemaphoreType.DMA,
        pltpu.SemaphoreType.DMA,
    ]

    @functools.partial(
        pl.kernel,
        mesh=mesh,
        out_type=jax.ShapeDtypeStruct((B, D), jnp.float32),
        scratch_types=scratch,
        compiler_params=pltpu.CompilerParams(use_tc_tiling_on_sc=False),
    )
    def k(head_hbm, tail_hbm, ids_hbm, out_hbm, ids_v, idx_a, idx_b,
          pos_a, pos_b, buf_a, buf_b, merged, sem_a, sem_b, sem_s):
        cid = lax.axis_index("c")
        sid = lax.axis_index("s")
        wid = cid * NS + sid
        base = wid * b_per_w
        pltpu.sync_copy(ids_hbm.at[pl.ds(base, b_per_w)], ids_v)

        lbase = sid * b_per_w
        trash_a = b_per_c + 2 * sid
        trash_b = trash_a + 1
        iota = lax.iota(jnp.int32, L)
        for i in range(b_per_w // L):
            v = ids_v[pl.ds(i * L, L)]
            m = v >= n_head
            idx_a[pl.ds(i * L, L)] = jnp.minimum(v, n_head - 1)
            idx_b[pl.ds(i * L, L)] = jnp.maximum(v - n_head, 0)
            rows = lbase + i * L + iota
            pos_a[pl.ds(i * L, L)] = jnp.where(m, trash_a, rows)
            pos_b[pl.ds(i * L, L)] = jnp.where(m, rows, trash_b)

        ga = pltpu.async_copy(head_hbm.at[idx_a], buf_a, sem_a)
        gb = pltpu.async_copy(tail_hbm.at[idx_b], buf_b, sem_b)
        ga.wait()
        sa = pltpu.async_copy(buf_a, merged.at[pos_a], sem_s)
        gb.wait()
        sb = pltpu.async_copy(buf_b, merged.at[pos_b], sem_s)
        sa.wait()
        sb.wait()
        plsc.subcore_barrier()

        pltpu.sync_copy(merged.at[pl.ds(lbase, b_per_w)], buf_a)
        pltpu.sync_copy(buf_a, out_hbm.at[pl.ds(base, b_per_w)])

    return k


def kernel(weight_head, trainable_buffer, input_ids):
    n_head, D = weight_head.shape
    n_tail = trainable_buffer.shape[0]
    B = input_ids.shape[0]
    k = _make_sc_gather(B, D, n_head, n_tail)
    return k(weight_head, trainable_buffer, input_ids.astype(jnp.int32))


# speedup vs baseline: 1.5031x; 1.4685x over previous
import functools

import jax
import jax.numpy as jnp
from jax import lax
from jax.experimental import pallas as pl
from jax.experimental.pallas import tpu as pltpu
from jax.experimental.pallas import tpu_sc as plsc

NC = 2
NS = 16
NW = NC * NS
L = 16


@functools.lru_cache(maxsize=None)
def _make_sc_gather(B, D, n_head, n_tail):
    assert B % (NW * L) == 0
    b_per_w = B // NW
    b_per_c = NS * b_per_w
    pad = b_per_w

    mesh = plsc.VectorSubcoreMesh(core_axis_name="c", subcore_axis_name="s")

    scratch = [
        pltpu.VMEM((b_per_w,), jnp.int32),
        pltpu.VMEM((b_per_w,), jnp.int32),
        pltpu.VMEM((b_per_w,), jnp.int32),
        pltpu.VMEM((b_per_w,), jnp.int32),
        pltpu.VMEM((b_per_w,), jnp.int32),
        pltpu.VMEM((b_per_w, D), jnp.float32),
        pltpu.VMEM((b_per_w, D), jnp.float32),
        pltpu.VMEM_SHARED((b_per_c + pad, D), jnp.float32),
        pltpu.SemaphoreType.DMA,
        pltpu.SemaphoreType.DMA,
        pltpu.SemaphoreType.DMA,
    ]

    @functools.partial(
        pl.kernel,
        mesh=mesh,
        out_type=jax.ShapeDtypeStruct((B, D), jnp.float32),
        scratch_types=scratch,
        compiler_params=pltpu.CompilerParams(use_tc_tiling_on_sc=False),
    )
    def k(head_hbm, tail_hbm, ids_hbm, out_hbm, ids_v, idx_a, idx_b,
          pos_a, pos_b, buf_a, buf_b, merged, sem_a, sem_b, sem_s):
        cid = lax.axis_index("c")
        sid = lax.axis_index("s")
        wid = cid * NS + sid
        base = wid * b_per_w
        pltpu.sync_copy(ids_hbm.at[pl.ds(base, b_per_w)], ids_v)

        lbase = sid * b_per_w
        iota = lax.iota(jnp.int32, L)
        for i in range(b_per_w // L):
            v = ids_v[pl.ds(i * L, L)]
            m = v >= n_head
            spread = (sid * (b_per_w // L) + i) * L + iota
            idx_a[pl.ds(i * L, L)] = jnp.where(m, spread, v)
            idx_b[pl.ds(i * L, L)] = jnp.where(m, v - n_head, spread % n_tail)
            rows = lbase + i * L + iota
            trash = b_per_c + (i * L + iota)
            pos_a[pl.ds(i * L, L)] = jnp.where(m, trash, rows)
            pos_b[pl.ds(i * L, L)] = jnp.where(m, rows, trash)

        ga = pltpu.async_copy(head_hbm.at[idx_a], buf_a, sem_a)
        gb = pltpu.async_copy(tail_hbm.at[idx_b], buf_b, sem_b)
        ga.wait()
        sa = pltpu.async_copy(buf_a, merged.at[pos_a], sem_s)
        gb.wait()
        sb = pltpu.async_copy(buf_b, merged.at[pos_b], sem_s)
        sa.wait()
        sb.wait()
        plsc.subcore_barrier()

        pltpu.sync_copy(merged.at[pl.ds(lbase, b_per_w)], buf_a)
        pltpu.sync_copy(buf_a, out_hbm.at[pl.ds(base, b_per_w)])

    return k


def kernel(weight_head, trainable_buffer, input_ids):
    n_head, D = weight_head.shape
    n_tail = trainable_buffer.shape[0]
    B = input_ids.shape[0]
    k = _make_sc_gather(B, D, n_head, n_tail)
    return k(weight_head, trainable_buffer, input_ids.astype(jnp.int32))
